# baseline (speedup 1.0000x reference)
import functools

import jax
import jax.numpy as jnp
from jax import lax
from jax.experimental import pallas as pl
from jax.experimental.pallas import tpu as pltpu
from jax.experimental.pallas import tpu_sc as plsc

N = 100000
D = 16
E = 3200000
NUM_LAYERS = 3
NUM_USERS = 60000
NUM_ITEMS = 40000

NC = 2
NS = 16
NW = NC * NS
BLK = 128
CB = 4
CHUNK_E = CB * BLK
PER_PAIR_E = 204800
E_PAD = PER_PAIR_E * NS
NCH = (384, 16)
assert NCH[0] % 4 == 0 and NCH[1] % 4 == 0 and NCH[0] + NCH[1] == PER_PAIR_E // CHUNK_E
_CORE0_BLKS = NS * NCH[0] * CB

N_PAD = 100096
ROWS_PER_TILE = N_PAD // NS

IDX_BYTES = CB * BLK * 4
DATA_BYTES = CHUNK_E * D * 4


def _lane_bcast(v16, e):
    return lax.gather(
        v16,
        jnp.full((16, 1), e, jnp.int32),
        lax.GatherDimensionNumbers(offset_dims=(), collapsed_slice_dims=(0,),
                                   start_index_map=(0,)),
        slice_sizes=(1,),
        mode=lax.GatherScatterMode.PROMISE_IN_BOUNDS,
    )


def _sc_propagate(table, rowb, colb, valb, zeros16):
    mesh = plsc.VectorSubcoreMesh(core_axis_name="c", subcore_axis_name="s")

    @functools.partial(
        pl.kernel,
        out_type=jax.ShapeDtypeStruct((NC, N_PAD, D), jnp.float32),
        mesh=mesh,
        scratch_types=[
            pltpu.VMEM_SHARED((N_PAD, D), jnp.float32),
            pltpu.VMEM((4, CB, BLK), jnp.int32),
            pltpu.VMEM((4, CB, BLK), jnp.int32),
            pltpu.VMEM((4, CB, BLK), jnp.float32),
            pltpu.VMEM((2, CHUNK_E, D), jnp.float32),
            [pltpu.SemaphoreType.DMA] * 4,
            [pltpu.SemaphoreType.DMA] * 2,
            [pltpu.SemaphoreType.DMA] * 2,
        ],
        compiler_params=pltpu.CompilerParams(use_tc_tiling_on_sc=False),
    )
    def k(table_hbm, rowb_hbm, colb_hbm, valb_hbm, zeros16_hbm, part_hbm,
          acc_sh, row_v, col_v, val_v, rows_v, sem_i, sem_g, sem_s):
        cid = lax.axis_index("c")
        sid = lax.axis_index("s")

        nch = jnp.where(cid == 0, NCH[0], NCH[1])
        blk0 = jnp.where(cid == 0, sid * (NCH[0] * CB),
                         _CORE0_BLKS + sid * (NCH[1] * CB))

        def fire_idx(c, m4):
            b = blk0 + c * CB
            pltpu.async_copy(colb_hbm.at[pl.ds(b, CB)], col_v.at[m4], sem_i[m4])
            pltpu.async_copy(rowb_hbm.at[pl.ds(b, CB)], row_v.at[m4], sem_i[m4])
            pltpu.async_copy(valb_hbm.at[pl.ds(b, CB)], val_v.at[m4], sem_i[m4])

        def wait_idx(m4):
            pltpu.make_async_copy(colb_hbm.at[pl.ds(0, CB)], col_v.at[m4], sem_i[m4]).wait()
            pltpu.make_async_copy(rowb_hbm.at[pl.ds(0, CB)], row_v.at[m4], sem_i[m4]).wait()
            pltpu.make_async_copy(valb_hbm.at[pl.ds(0, CB)], val_v.at[m4], sem_i[m4]).wait()

        def fire_gathers(m4, p2):
            for j in range(CB):
                pltpu.async_copy(table_hbm.at[col_v.at[m4, j]],
                                 rows_v.at[p2, pl.ds(j * BLK, BLK)], sem_g[p2])

        def drain_gathers(p2):
            pltpu.make_async_copy(table_hbm.at[pl.ds(0, CHUNK_E)],
                                  rows_v.at[p2], sem_g[p2]).wait()

        def scale(p2, m4):
            @pl.loop(0, CB * BLK // 16)
            def _grp(g):
                r = g // 8
                o = (g % 8) * 16
                v16 = val_v[m4, r, pl.ds(o, 16)]
                for e in range(16):
                    bc = _lane_bcast(v16, e)
                    i = g * 16 + e
                    rows_v[p2, i, :] = rows_v[p2, i, :] * bc

        def fire_scatters(m4, p2):
            for j in range(CB):
                pltpu.async_copy(rows_v.at[p2, pl.ds(j * BLK, BLK)],
                                 acc_sh.at[row_v.at[m4, j]], sem_s[p2], add=True)

        def drain_scatters(p2):
            pltpu.make_async_copy(table_hbm.at[pl.ds(0, CHUNK_E)],
                                  rows_v.at[p2], sem_s[p2]).wait()

        zbase = sid * ROWS_PER_TILE
        pltpu.sync_copy(zeros16_hbm.at[pl.ds(zbase, ROWS_PER_TILE)],
                        acc_sh.at[pl.ds(zbase, ROWS_PER_TILE)])

        fire_idx(0, 0)
        wait_idx(0)
        fire_gathers(0, 0)
        fire_idx(1, 1)
        plsc.subcore_barrier()

        wait_idx(1)
        fire_gathers(1, 1)
        fire_idx(2, 2)
        drain_gathers(0)
        scale(0, 0)
        fire_scatters(0, 0)

        @pl.loop(0, (nch - 4) // 4)
        def _steady(kk):
            c0 = 2 + kk * 4
            for i in range(4):
                c = c0 + i
                p2 = i % 2
                p4 = (2 + i) % 4
                q4 = (3 + i) % 4
                r4 = (1 + i) % 4
                drain_scatters(p2)
                wait_idx(p4)
                fire_gathers(p4, p2)
                fire_idx(c + 1, q4)
                drain_gathers(p2 ^ 1)
                scale(p2 ^ 1, r4)
                fire_scatters(r4, p2 ^ 1)

        drain_scatters(0)
        wait_idx(2)
        fire_gathers(2, 0)
        fire_idx(nch - 1, 3)
        drain_gathers(1)
        scale(1, 1)
        fire_scatters(1, 1)
        drain_scatters(1)
        wait_idx(3)
        fire_gathers(3, 1)
        drain_gathers(0)
        scale(0, 2)
        fire_scatters(2, 0)
        drain_scatters(0)
        drain_gathers(1)
        scale(1, 3)
        fire_scatters(3, 1)
        drain_scatters(1)

        plsc.subcore_barrier()
        pltpu.sync_copy(acc_sh.at[pl.ds(zbase, ROWS_PER_TILE)],
                        part_hbm.at[cid, pl.ds(zbase, ROWS_PER_TILE)])

    return k(table, rowb, colb, valb, zeros16)


_CROWS = N_PAD * D // 128
_CBLK = 544


def _tc_combine(part, msum, scale):
    p = part.reshape(NC, _CROWS, 128)
    m = msum.reshape(_CROWS, 128)

    def body(p_ref, m_ref, ne_ref, nm_ref):
        s = p_ref[0] + p_ref[1]
        ne_ref[...] = s
        nm_ref[...] = (m_ref[...] + s) * scale

    ne, nm = pl.pallas_call(
        body,
        grid=(_CROWS // _CBLK,),
        in_specs=[pl.BlockSpec((NC, _CBLK, 128), lambda i: (0, i, 0)),
                  pl.BlockSpec((_CBLK, 128), lambda i: (i, 0))],
        out_specs=[pl.BlockSpec((_CBLK, 128), lambda i: (i, 0)),
                   pl.BlockSpec((_CBLK, 128), lambda i: (i, 0))],
        out_shape=[jax.ShapeDtypeStruct((_CROWS, 128), jnp.float32)] * 2,
    )(p, m)
    return ne.reshape(N_PAD, D), nm.reshape(N_PAD, D)


def kernel(emb, val, row, col):
    emb_p = jnp.concatenate([emb, jnp.zeros((N_PAD - N, D), jnp.float32)])
    pad = E_PAD - E
    filli = jnp.full((pad,), N, jnp.int32)
    rowb = jnp.concatenate([row, filli]).reshape(E_PAD // BLK, BLK)
    colb = jnp.concatenate([col, filli]).reshape(E_PAD // BLK, BLK)
    valb = jnp.concatenate([val, jnp.zeros((pad,), jnp.float32)]).reshape(E_PAD // BLK, BLK)
    zeros16 = jnp.zeros((N_PAD, D), jnp.float32)

    cur = emb_p
    msum = emb_p
    for layer in range(NUM_LAYERS):
        part = _sc_propagate(cur, rowb, colb, valb, zeros16)
        scale = 0.25 if layer == NUM_LAYERS - 1 else 1.0
        cur, msum = _tc_combine(part, msum, scale)

    light = msum[:N]
    return (light[:NUM_USERS], light[NUM_USERS:NUM_USERS + NUM_ITEMS])


# speedup vs baseline: 1.0717x; 1.0717x over previous
import functools

import jax
import jax.numpy as jnp
from jax import lax
from jax.experimental import pallas as pl
from jax.experimental.pallas import tpu as pltpu
from jax.experimental.pallas import tpu_sc as plsc

N = 100000
D = 16
E = 3200000
NUM_LAYERS = 3
NUM_USERS = 60000
NUM_ITEMS = 40000

NC = 2
NS = 16
NW = NC * NS
BLK = 128
CB = 4
CHUNK_E = CB * BLK
PER_PAIR_E = 204800
E_PAD = PER_PAIR_E * NS
NCH = (368, 32)
assert NCH[0] % 4 == 0 and NCH[1] % 4 == 0 and NCH[0] + NCH[1] == PER_PAIR_E // CHUNK_E
_CORE0_BLKS = NS * NCH[0] * CB

N_PAD = 100096
ROWS_PER_TILE = N_PAD // NS

IDX_BYTES = CB * BLK * 4
DATA_BYTES = CHUNK_E * D * 4


def _lane_bcast(v16, e):
    return lax.gather(
        v16,
        jnp.full((16, 1), e, jnp.int32),
        lax.GatherDimensionNumbers(offset_dims=(), collapsed_slice_dims=(0,),
                                   start_index_map=(0,)),
        slice_sizes=(1,),
        mode=lax.GatherScatterMode.PROMISE_IN_BOUNDS,
    )


def _sc_propagate(table, rowb, colb, valb, zeros16):
    mesh = plsc.VectorSubcoreMesh(core_axis_name="c", subcore_axis_name="s")

    @functools.partial(
        pl.kernel,
        out_type=jax.ShapeDtypeStruct((NC, N_PAD, D), jnp.float32),
        mesh=mesh,
        scratch_types=[
            pltpu.VMEM_SHARED((N_PAD, D), jnp.float32),
            pltpu.VMEM((4, CB, BLK), jnp.int32),
            pltpu.VMEM((4, CB, BLK), jnp.int32),
            pltpu.VMEM((4, CB, BLK), jnp.float32),
            pltpu.VMEM((2, CHUNK_E, D), jnp.float32),
            [pltpu.SemaphoreType.DMA] * 4,
            [pltpu.SemaphoreType.DMA] * 2,
            [pltpu.SemaphoreType.DMA] * 2,
        ],
        compiler_params=pltpu.CompilerParams(use_tc_tiling_on_sc=False),
    )
    def k(table_hbm, rowb_hbm, colb_hbm, valb_hbm, zeros16_hbm, part_hbm,
          acc_sh, row_v, col_v, val_v, rows_v, sem_i, sem_g, sem_s):
        cid = lax.axis_index("c")
        sid = lax.axis_index("s")

        nch = jnp.where(cid == 0, NCH[0], NCH[1])
        blk0 = jnp.where(cid == 0, sid * (NCH[0] * CB),
                         _CORE0_BLKS + sid * (NCH[1] * CB))

        def fire_idx(c, m4):
            b = blk0 + c * CB
            pltpu.async_copy(colb_hbm.at[pl.ds(b, CB)], col_v.at[m4], sem_i[m4])
            pltpu.async_copy(rowb_hbm.at[pl.ds(b, CB)], row_v.at[m4], sem_i[m4])
            pltpu.async_copy(valb_hbm.at[pl.ds(b, CB)], val_v.at[m4], sem_i[m4])

        def wait_idx(m4):
            pltpu.make_async_copy(colb_hbm.at[pl.ds(0, CB)], col_v.at[m4], sem_i[m4]).wait()
            pltpu.make_async_copy(rowb_hbm.at[pl.ds(0, CB)], row_v.at[m4], sem_i[m4]).wait()
            pltpu.make_async_copy(valb_hbm.at[pl.ds(0, CB)], val_v.at[m4], sem_i[m4]).wait()

        def fire_gathers(m4, p2):
            for j in range(CB):
                pltpu.async_copy(table_hbm.at[col_v.at[m4, j]],
                                 rows_v.at[p2, pl.ds(j * BLK, BLK)], sem_g[p2])

        def drain_gathers(p2):
            pltpu.make_async_copy(table_hbm.at[pl.ds(0, CHUNK_E)],
                                  rows_v.at[p2], sem_g[p2]).wait()

        def scale(p2, m4):
            @pl.loop(0, CB * BLK // 16)
            def _grp(g):
                r = g // 8
                o = (g % 8) * 16
                v16 = val_v[m4, r, pl.ds(o, 16)]
                for e in range(16):
                    bc = _lane_bcast(v16, e)
                    i = g * 16 + e
                    rows_v[p2, i, :] = rows_v[p2, i, :] * bc

        def fire_scatters(m4, p2):
            for j in range(CB):
                pltpu.async_copy(rows_v.at[p2, pl.ds(j * BLK, BLK)],
                                 acc_sh.at[row_v.at[m4, j]], sem_s[p2], add=True)

        def drain_scatters(p2):
            pltpu.make_async_copy(table_hbm.at[pl.ds(0, CHUNK_E)],
                                  rows_v.at[p2], sem_s[p2]).wait()

        zbase = sid * ROWS_PER_TILE
        pltpu.sync_copy(zeros16_hbm.at[pl.ds(zbase, ROWS_PER_TILE)],
                        acc_sh.at[pl.ds(zbase, ROWS_PER_TILE)])

        fire_idx(0, 0)
        wait_idx(0)
        fire_gathers(0, 0)
        fire_idx(1, 1)
        plsc.subcore_barrier()

        wait_idx(1)
        fire_gathers(1, 1)
        fire_idx(2, 2)
        drain_gathers(0)
        scale(0, 0)
        fire_scatters(0, 0)

        @pl.loop(0, (nch - 4) // 4)
        def _steady(kk):
            c0 = 2 + kk * 4
            for i in range(4):
                c = c0 + i
                p2 = i % 2
                p4 = (2 + i) % 4
                q4 = (3 + i) % 4
                r4 = (1 + i) % 4
                drain_scatters(p2)
                wait_idx(p4)
                fire_gathers(p4, p2)
                fire_idx(c + 1, q4)
                drain_gathers(p2 ^ 1)
                scale(p2 ^ 1, r4)
                fire_scatters(r4, p2 ^ 1)

        drain_scatters(0)
        wait_idx(2)
        fire_gathers(2, 0)
        fire_idx(nch - 1, 3)
        drain_gathers(1)
        scale(1, 1)
        fire_scatters(1, 1)
        drain_scatters(1)
        wait_idx(3)
        fire_gathers(3, 1)
        drain_gathers(0)
        scale(0, 2)
        fire_scatters(2, 0)
        drain_scatters(0)
        drain_gathers(1)
        scale(1, 3)
        fire_scatters(3, 1)
        drain_scatters(1)

        plsc.subcore_barrier()
        pltpu.sync_copy(acc_sh.at[pl.ds(zbase, ROWS_PER_TILE)],
                        part_hbm.at[cid, pl.ds(zbase, ROWS_PER_TILE)])

    return k(table, rowb, colb, valb, zeros16)


_CROWS = N_PAD * D // 128
_CBLK = 544


def _tc_combine(part, msum, scale):
    p = part.reshape(NC, _CROWS, 128)
    m = msum.reshape(_CROWS, 128)

    def body(p_ref, m_ref, ne_ref, nm_ref):
        s = p_ref[0] + p_ref[1]
        ne_ref[...] = s
        nm_ref[...] = (m_ref[...] + s) * scale

    ne, nm = pl.pallas_call(
        body,
        grid=(_CROWS // _CBLK,),
        in_specs=[pl.BlockSpec((NC, _CBLK, 128), lambda i: (0, i, 0)),
                  pl.BlockSpec((_CBLK, 128), lambda i: (i, 0))],
        out_specs=[pl.BlockSpec((_CBLK, 128), lambda i: (i, 0)),
                   pl.BlockSpec((_CBLK, 128), lambda i: (i, 0))],
        out_shape=[jax.ShapeDtypeStruct((_CROWS, 128), jnp.float32)] * 2,
    )(p, m)
    return ne.reshape(N_PAD, D), nm.reshape(N_PAD, D)


def kernel(emb, val, row, col):
    emb_p = jnp.concatenate([emb, jnp.zeros((N_PAD - N, D), jnp.float32)])
    pad = E_PAD - E
    filli = jnp.full((pad,), N, jnp.int32)
    rowb = jnp.concatenate([row, filli]).reshape(E_PAD // BLK, BLK)
    colb = jnp.concatenate([col, filli]).reshape(E_PAD // BLK, BLK)
    valb = jnp.concatenate([val, jnp.zeros((pad,), jnp.float32)]).reshape(E_PAD // BLK, BLK)
    zeros16 = jnp.zeros((N_PAD, D), jnp.float32)

    cur = emb_p
    msum = emb_p
    for layer in range(NUM_LAYERS):
        part = _sc_propagate(cur, rowb, colb, valb, zeros16)
        scale = 0.25 if layer == NUM_LAYERS - 1 else 1.0
        cur, msum = _tc_combine(part, msum, scale)

    light = msum[:N]
    return (light[:NUM_USERS], light[NUM_USERS:NUM_USERS + NUM_ITEMS])
